# baseline (speedup 1.0000x reference)
import functools

import jax
import jax.numpy as jnp
from jax import lax
from jax.experimental import pallas as pl
from jax.experimental.pallas import tpu as pltpu
from jax.experimental.pallas import tpu_sc as plsc

_VOCAB = 1000000
_EMBED_DIM = 64
_BATCH = 16384
_HIST = 50

_NC = 2
_NS = 16
_NW = _NC * _NS
_BB = 128
_NBLK = _BATCH // _BB
_KPW = _NBLK // _NW
_ITERS = _KPW * _HIST


def _transpose_block(gbuf, tbuf):
    iota = lax.iota(jnp.int32, 16)
    zeros = jnp.zeros((16,), jnp.int32)
    bases = [lax.shift_left(iota + 16 * j, 6) for j in range(8)]

    @plsc.parallel_loop(0, _EMBED_DIM, unroll=4)
    def _(e):
        eb = lax.shift_right_logical(e, 3)
        ep = lax.bitwise_and(e, 7)
        vs = [plsc.load_gather(gbuf, [zeros, bases[j] + e]) for j in range(8)]
        for j in range(8):
            tbuf[eb, ep, pl.ds(16 * j, 16)] = vs[j]


def _embed_kernel(idx_hbm, w_hbm, l_hbm,
                  idx_v, gb0, gb1, tb0, tb1,
                  gsem0, gsem1, wsem0, wsem1):
    wid = lax.axis_index("s") * _NC + lax.axis_index("c")
    gbs, tbs = (gb0, gb1), (tb0, tb1)
    gsems, wsems = (gsem0, gsem1), (wsem0, wsem1)

    pltpu.sync_copy(idx_hbm.at[wid], idx_v)

    def fire_gather(i, p):
        pltpu.async_copy(w_hbm.at[idx_v.at[i]], gbs[p], gsems[p])

    fire_gather(0, 0)
    fire_gather(1, 1)

    def step(i2, carry):
        for p in range(2):
            i = 2 * i2 + p
            k = i // _HIST
            h = i - k * _HIST
            bb = _KPW * wid + k
            pltpu.make_async_copy(
                w_hbm.at[pl.ds(0, _BB)], gbs[p], gsems[p]
            ).wait()

            @pl.when(i2 >= 1)
            def _():
                pltpu.make_async_copy(
                    tbs[p], l_hbm.at[0, :, 0], wsems[p]
                ).wait()

            _transpose_block(gbs[p], tbs[p])

            pltpu.async_copy(tbs[p], l_hbm.at[h, :, bb], wsems[p])

            @pl.when(i + 2 < _ITERS)
            def _():
                fire_gather(i + 2, p)

        return carry

    lax.fori_loop(0, _ITERS // 2, step, 0)

    for p in range(2):
        pltpu.make_async_copy(tbs[p], l_hbm.at[0, :, 0], wsems[p]).wait()


@jax.jit
def _embed(idxP, w):
    mesh = plsc.VectorSubcoreMesh(core_axis_name="c", subcore_axis_name="s")
    f = functools.partial(
        pl.kernel,
        mesh=mesh,
        out_type=jax.ShapeDtypeStruct((_HIST, 8, _NBLK, 8, _BB), jnp.float32),
        scratch_types=[
            pltpu.VMEM((_ITERS, _BB), jnp.int32),
            pltpu.VMEM((_BB, _EMBED_DIM), jnp.float32),
            pltpu.VMEM((_BB, _EMBED_DIM), jnp.float32),
            pltpu.VMEM((8, 8, _BB), jnp.float32),
            pltpu.VMEM((8, 8, _BB), jnp.float32),
            pltpu.SemaphoreType.DMA,
            pltpu.SemaphoreType.DMA,
            pltpu.SemaphoreType.DMA,
            pltpu.SemaphoreType.DMA,
        ],
        compiler_params=pltpu.CompilerParams(
            use_tc_tiling_on_sc=False, needs_layout_passes=False
        ),
    )(_embed_kernel)
    return f(idxP, w)


def kernel(tokenIndex, weights):
    idx = tokenIndex.astype(jnp.int32)
    idxP = (idx.T.reshape(_HIST, _NBLK, _BB).transpose(1, 0, 2)
            .reshape(_NW, _ITERS, _BB))
    L = _embed(idxP, weights)
    return (
        L.transpose(2, 4, 0, 1, 3)
        .reshape(_BATCH, _HIST, _EMBED_DIM)
    )


# speedup vs baseline: 1.1933x; 1.1933x over previous
import functools

import jax
import jax.numpy as jnp
from jax import lax
from jax.experimental import pallas as pl
from jax.experimental.pallas import tpu as pltpu
from jax.experimental.pallas import tpu_sc as plsc

_VOCAB = 1000000
_EMBED_DIM = 64
_BATCH = 16384
_HIST = 50

_NC = 2
_NS = 16
_NW = _NC * _NS
_BB = 128
_NBLK = _BATCH // _BB
_KPW = _NBLK // _NW
_ITERS = _KPW * _HIST
_NTILE = _HIST * (_EMBED_DIM // 8) * _NBLK


def _transpose_block(gbuf, tbuf):
    iota = lax.iota(jnp.int32, 16)
    zeros = jnp.zeros((16,), jnp.int32)
    bases = [lax.shift_left(iota + 16 * j, 6) for j in range(8)]

    @plsc.parallel_loop(0, _EMBED_DIM, unroll=4)
    def _(e):
        for j in range(8):
            v = plsc.load_gather(gbuf, [zeros, bases[j] + e])
            tbuf[e, pl.ds(16 * j, 16)] = v


def _embed_kernel(idx_hbm, w_hbm, l_hbm,
                  idx_v, gb0, gb1, tb0, tb1,
                  gsem0, gsem1, wsem0, wsem1):
    wid = lax.axis_index("s") * _NC + lax.axis_index("c")
    gbs, tbs = (gb0, gb1), (tb0, tb1)
    gsems, wsems = (gsem0, gsem1), (wsem0, wsem1)

    pltpu.sync_copy(idx_hbm.at[wid], idx_v)

    def fire_gather(i, p):
        pltpu.async_copy(w_hbm.at[idx_v.at[i]], gbs[p], gsems[p])

    fire_gather(0, 0)
    fire_gather(1, 1)

    def step(i2, carry):
        for p in range(2):
            i = 2 * i2 + p
            k = i // _HIST
            h = i - k * _HIST
            bb = _KPW * wid + k
            pltpu.make_async_copy(
                w_hbm.at[pl.ds(0, _BB)], gbs[p], gsems[p]
            ).wait()

            @pl.when(i2 >= 1)
            def _():
                for _eb in range(8):
                    pltpu.make_async_copy(
                        tbs[p].at[pl.ds(8 * _eb, 8)], l_hbm.at[0], wsems[p]
                    ).wait()

            _transpose_block(gbs[p], tbs[p])

            for eb in range(8):
                pltpu.async_copy(
                    tbs[p].at[pl.ds(8 * eb, 8)],
                    l_hbm.at[h * 1024 + eb * 128 + bb],
                    wsems[p],
                )

            @pl.when(i + 2 < _ITERS)
            def _():
                fire_gather(i + 2, p)

        return carry

    lax.fori_loop(0, _ITERS // 2, step, 0)

    for p in range(2):
        for _eb in range(8):
            pltpu.make_async_copy(
                tbs[p].at[pl.ds(8 * _eb, 8)], l_hbm.at[0], wsems[p]
            ).wait()


@jax.jit
def _embed(idxP, w):
    mesh = plsc.VectorSubcoreMesh(core_axis_name="c", subcore_axis_name="s")
    f = functools.partial(
        pl.kernel,
        mesh=mesh,
        out_type=jax.ShapeDtypeStruct((_NTILE, 8, 128), jnp.float32),
        scratch_types=[
            pltpu.VMEM((_ITERS, _BB), jnp.int32),
            pltpu.VMEM((_BB, _EMBED_DIM), jnp.float32),
            pltpu.VMEM((_BB, _EMBED_DIM), jnp.float32),
            pltpu.VMEM((_EMBED_DIM, _BB), jnp.float32),
            pltpu.VMEM((_EMBED_DIM, _BB), jnp.float32),
            pltpu.SemaphoreType.DMA,
            pltpu.SemaphoreType.DMA,
            pltpu.SemaphoreType.DMA,
            pltpu.SemaphoreType.DMA,
        ],
        compiler_params=pltpu.CompilerParams(
            use_tc_tiling_on_sc=False, needs_layout_passes=False
        ),
    )(_embed_kernel)
    return f(idxP, w)


def kernel(tokenIndex, weights):
    idx = tokenIndex.astype(jnp.int32)
    idxP = (idx.T.reshape(_HIST, _NBLK, _BB).transpose(1, 0, 2)
            .reshape(_NW, _ITERS, _BB))
    L = _embed(idxP, weights)
    return (
        L.reshape(_HIST, 8, _NBLK, 8, _BB)
        .transpose(2, 4, 0, 1, 3)
        .reshape(_BATCH, _HIST, _EMBED_DIM)
    )
